# Initial kernel scaffold; baseline (speedup 1.0000x reference)
import jax
import jax.numpy as jnp
from jax.experimental import pallas as pl


def kernel(query_x, query_pos, src_x_0, src_pos_0, src_x_1, src_pos_1, edge_src_0, edge_dst_0, edge_src_1, edge_dst_1, W_pre_0, b_pre_0, W_pre_1, b_pre_1, Wq, Wk, Wv, Wo, W_gate):
    raise NotImplementedError("write your pallas kernel here")


# speedup vs baseline: 24.3007x; 24.3007x over previous
import functools

import jax
import jax.numpy as jnp
import numpy as np
from jax import lax
from jax.experimental import pallas as pl
from jax.experimental.pallas import tpu as pltpu
from jax.experimental.pallas import tpu_sc as plsc

H = 4
DH = 32
LEN = 32
R0 = 1.0
R1 = 2.0

NC = 2
NS = 16
NW = NC * NS
CH = 80


def _proj_kv_body(x_ref, wk_ref, wv_ref, k_ref, v_ref):
    x = x_ref[...]
    k_ref[...] = jnp.dot(x, wk_ref[...], preferred_element_type=jnp.float32)
    v_ref[...] = jnp.dot(x, wv_ref[...], preferred_element_type=jnp.float32)


def _proj_q_body(x_ref, wq_ref, q_ref):
    q_ref[...] = jnp.dot(x_ref[...], wq_ref[...],
                         preferred_element_type=jnp.float32)


def _proj_kv(src_x, Wk, Wv, blk=1000):
    n = src_x.shape[0]
    d = src_x.shape[1]
    grid = n // blk
    return pl.pallas_call(
        _proj_kv_body,
        grid=(grid,),
        in_specs=[
            pl.BlockSpec((blk, d), lambda i: (i, 0)),
            pl.BlockSpec((d, d), lambda i: (0, 0)),
            pl.BlockSpec((d, d), lambda i: (0, 0)),
        ],
        out_specs=[
            pl.BlockSpec((blk, d), lambda i: (i, 0)),
            pl.BlockSpec((blk, d), lambda i: (i, 0)),
        ],
        out_shape=[
            jax.ShapeDtypeStruct((n, d), jnp.float32),
            jax.ShapeDtypeStruct((n, d), jnp.float32),
        ],
    )(src_x, Wk, Wv)


def _proj_q(query_x, Wq, blk=2000):
    n, d = query_x.shape
    return pl.pallas_call(
        _proj_q_body,
        grid=(n // blk,),
        in_specs=[
            pl.BlockSpec((blk, d), lambda i: (i, 0)),
            pl.BlockSpec((d, d), lambda i: (0, 0)),
        ],
        out_specs=pl.BlockSpec((blk, d), lambda i: (i, 0)),
        out_shape=jax.ShapeDtypeStruct((n, d), jnp.float32),
    )(query_x, Wq)


def _make_gather(E_tot, NQ, NSRC, D):
    per_w = E_tot // NW
    n_chunks = per_w // CH
    mesh = plsc.VectorSubcoreMesh(core_axis_name="c", subcore_axis_name="s")

    @functools.partial(
        pl.kernel,
        mesh=mesh,
        out_type=[
            jax.ShapeDtypeStruct((E_tot, D), jnp.float32),
            jax.ShapeDtypeStruct((E_tot, D), jnp.float32),
            jax.ShapeDtypeStruct((E_tot, D), jnp.float32),
            jax.ShapeDtypeStruct((E_tot, 128), jnp.float32),
            jax.ShapeDtypeStruct((E_tot, 128), jnp.float32),
        ],
        scratch_types=[
            pltpu.VMEM((CH,), jnp.int32),
            pltpu.VMEM((CH,), jnp.int32),
            pltpu.VMEM((CH, D), jnp.float32),
            pltpu.VMEM((CH, D), jnp.float32),
            pltpu.VMEM((CH, D), jnp.float32),
            pltpu.VMEM((CH, 128), jnp.float32),
            pltpu.VMEM((CH, 128), jnp.float32),
            pltpu.SemaphoreType.DMA,
        ],
    )
    def gather(e_src_hbm, e_dst_hbm, k_hbm, v_hbm, q_hbm, ps_hbm, pq_hbm,
               ke_out, ve_out, qe_out, pse_out, pqe_out,
               idxs_v, idxd_v, kbuf, vbuf, qbuf, psbuf, pqbuf, sem):
        wid = lax.axis_index("s") * NC + lax.axis_index("c")

        def body(j, carry):
            base = wid * per_w + j * CH
            pltpu.sync_copy(e_src_hbm.at[pl.ds(base, CH)], idxs_v)
            pltpu.sync_copy(e_dst_hbm.at[pl.ds(base, CH)], idxd_v)
            cps = [
                pltpu.async_copy(k_hbm.at[idxs_v], kbuf, sem),
                pltpu.async_copy(v_hbm.at[idxs_v], vbuf, sem),
                pltpu.async_copy(q_hbm.at[idxd_v], qbuf, sem),
                pltpu.async_copy(ps_hbm.at[idxs_v], psbuf, sem),
                pltpu.async_copy(pq_hbm.at[idxd_v], pqbuf, sem),
            ]
            for cp in cps:
                cp.wait()
            pltpu.sync_copy(kbuf, ke_out.at[pl.ds(base, CH)])
            pltpu.sync_copy(vbuf, ve_out.at[pl.ds(base, CH)])
            pltpu.sync_copy(qbuf, qe_out.at[pl.ds(base, CH)])
            pltpu.sync_copy(psbuf, pse_out.at[pl.ds(base, CH)])
            pltpu.sync_copy(pqbuf, pqe_out.at[pl.ds(base, CH)])
            return carry

        lax.fori_loop(0, n_chunks, body, 0)

    return gather


def _edge_body(nblk0, rsqrt_dh,
               qe_ref, ke_ref, ve_ref, ps_ref, pq_ref,
               w0_ref, w1_ref, b0_ref, b1_ref, wg_ref,
               sel_ref, selt_ref, p16_ref,
               msg_ref, ex_ref):
    is0 = pl.program_id(0) < nblk0
    r_cut = jnp.where(is0, R0, R1)

    rel = ps_ref[...] - pq_ref[...]
    l2 = jnp.sum(rel * rel, axis=1, keepdims=True) + 1e-12
    length = jnp.sqrt(l2)

    centers = lax.broadcasted_iota(jnp.int32, (1, LEN), 1).astype(
        jnp.float32) * (r_cut / (LEN - 1))
    sigma = r_cut / LEN
    z = (length - centers) / sigma
    emb = jnp.exp(-0.5 * z * z)

    W_pre = jnp.where(is0, w0_ref[...], w1_ref[...])
    b_pre = jnp.where(is0, b0_ref[...], b1_ref[...])
    pre = jnp.dot(emb, W_pre, preferred_element_type=jnp.float32) + b_pre
    es = pre * jax.nn.sigmoid(pre)
    gate = jnp.dot(es, wg_ref[...], preferred_element_type=jnp.float32)

    prod = qe_ref[...] * ke_ref[...]
    qk = jnp.dot(prod, sel_ref[...],
                 preferred_element_type=jnp.float32)

    env = 0.5 * (jnp.cos(np.pi * jnp.clip(length / r_cut, 0.0, 1.0)) + 1.0)
    ex = jnp.exp(qk * rsqrt_dh + gate) * (env + 1e-6)

    ex_ref[...] = jnp.dot(ex, p16_ref[...],
                          preferred_element_type=jnp.float32)
    msg_ref[...] = jnp.dot(ex, selt_ref[...],
                           preferred_element_type=jnp.float32) * ve_ref[...]


def _edge_math(qe, ke, ve, pse, pqe, W0, W1, b0, b1, Wg16, sel, selt, p16,
               E_per_scale, blk=2000):
    E_tot = qe.shape[0]
    D = qe.shape[1]
    nblk0 = E_per_scale // blk
    grid = E_tot // blk
    body = functools.partial(_edge_body, nblk0, float(1.0 / np.sqrt(DH)))
    row = lambda i: (i, 0)
    full = lambda i: (0, 0)
    return pl.pallas_call(
        body,
        grid=(grid,),
        in_specs=[
            pl.BlockSpec((blk, D), row),
            pl.BlockSpec((blk, D), row),
            pl.BlockSpec((blk, D), row),
            pl.BlockSpec((blk, 128), row),
            pl.BlockSpec((blk, 128), row),
            pl.BlockSpec((LEN, LEN), full),
            pl.BlockSpec((LEN, LEN), full),
            pl.BlockSpec((1, LEN), full),
            pl.BlockSpec((1, LEN), full),
            pl.BlockSpec((LEN, 16), full),
            pl.BlockSpec((D, 16), full),
            pl.BlockSpec((16, D), full),
            pl.BlockSpec((16, D), full),
        ],
        out_specs=[
            pl.BlockSpec((blk, D), row),
            pl.BlockSpec((blk, D), row),
        ],
        out_shape=[
            jax.ShapeDtypeStruct((E_tot, D), jnp.float32),
            jax.ShapeDtypeStruct((E_tot, D), jnp.float32),
        ],
    )(qe, ke, ve, pse, pqe, W0, W1, b0, b1, Wg16, sel, selt, p16)


def _make_scatter(E_tot, NQ, D):
    per_w = E_tot // NW
    n_chunks = per_w // CH
    ZR = 200
    n_zc = NQ // ZR
    mesh = plsc.VectorSubcoreMesh(core_axis_name="c", subcore_axis_name="s")

    @functools.partial(
        pl.kernel,
        mesh=mesh,
        out_type=jax.ShapeDtypeStruct((NC * NQ, D), jnp.float32),
        scratch_types=[
            pltpu.VMEM((CH,), jnp.int32),
            pltpu.VMEM((CH, D), jnp.float32),
            pltpu.VMEM((ZR, D), jnp.float32),
            pltpu.VMEM_SHARED((NQ, D), jnp.float32),
        ],
    )
    def scatter(e_dst_hbm, val_hbm, part_out, idx_v, vbuf, zrow, sh):
        c = lax.axis_index("c")
        s = lax.axis_index("s")
        wid = s * NC + c

        zero16 = jnp.zeros((16,), jnp.float32)

        def zb(i, carry):
            r = i // (D // 16)
            col = (i % (D // 16)) * 16
            zrow[r, pl.ds(col, 16)] = zero16
            return carry
        lax.fori_loop(0, ZR * (D // 16), zb, 0)

        def zfill(t, carry):
            cid = s + NS * t

            @pl.when(cid < n_zc)
            def _():
                pltpu.sync_copy(zrow, sh.at[pl.ds(cid * ZR, ZR)])
            return carry
        lax.fori_loop(0, (n_zc + NS - 1) // NS, zfill, 0)

        plsc.subcore_barrier()

        def body(j, carry):
            base = wid * per_w + j * CH
            pltpu.sync_copy(e_dst_hbm.at[pl.ds(base, CH)], idx_v)
            pltpu.sync_copy(val_hbm.at[pl.ds(base, CH)], vbuf)
            pltpu.sync_copy(vbuf, sh.at[idx_v], add=True)
            return carry

        lax.fori_loop(0, n_chunks, body, 0)

        plsc.subcore_barrier()

        def wb(t, carry):
            cid = s + NS * t

            @pl.when(cid < n_zc)
            def _():
                r0 = cid * ZR
                pltpu.sync_copy(sh.at[pl.ds(r0, ZR)],
                                part_out.at[pl.ds(c * NQ + r0, ZR)])
            return carry
        lax.fori_loop(0, (n_zc + NS - 1) // NS, wb, 0)

    return scatter


def _final_body(qx_ref, n0_ref, n1_ref, d0_ref, d1_ref, s16_ref, wo_ref,
                out_ref):
    num = n0_ref[...] + n1_ref[...]
    den = jnp.dot((d0_ref[...] + d1_ref[...])[:, :16], s16_ref[...],
                  preferred_element_type=jnp.float32) + 1e-9
    agg = num / den
    out_ref[...] = jnp.dot(agg, wo_ref[...],
                           preferred_element_type=jnp.float32) + qx_ref[...]


def _finalize(query_x, nparts, dparts, S16, Wo, blk=2000):
    NQ, D = query_x.shape
    nb = NQ // blk
    return pl.pallas_call(
        _final_body,
        grid=(nb,),
        in_specs=[
            pl.BlockSpec((blk, D), lambda i: (i, 0)),
            pl.BlockSpec((blk, D), lambda i: (i, 0)),
            pl.BlockSpec((blk, D), lambda i, nb=nb: (i + nb, 0)),
            pl.BlockSpec((blk, D), lambda i: (i, 0)),
            pl.BlockSpec((blk, D), lambda i, nb=nb: (i + nb, 0)),
            pl.BlockSpec((16, D), lambda i: (0, 0)),
            pl.BlockSpec((D, D), lambda i: (0, 0)),
        ],
        out_specs=pl.BlockSpec((blk, D), lambda i: (i, 0)),
        out_shape=jax.ShapeDtypeStruct((NQ, D), jnp.float32),
    )(query_x, nparts, nparts, dparts, dparts, S16, Wo)


def kernel(query_x, query_pos, src_x_0, src_pos_0, src_x_1, src_pos_1,
           edge_src_0, edge_dst_0, edge_src_1, edge_dst_1,
           W_pre_0, b_pre_0, W_pre_1, b_pre_1, Wq, Wk, Wv, Wo, W_gate):
    NQ, D = query_x.shape
    NS0 = src_x_0.shape[0]
    NS1 = src_x_1.shape[0]
    E = edge_src_0.shape[0]
    E_tot = 2 * E

    src_x = jnp.concatenate([src_x_0, src_x_1], axis=0)
    e_src = jnp.concatenate([edge_src_0, edge_src_1 + NS0], axis=0)
    e_dst = jnp.concatenate([edge_dst_0, edge_dst_1], axis=0)
    src_pos = jnp.pad(jnp.concatenate([src_pos_0, src_pos_1], axis=0),
                      ((0, 0), (0, 125)))
    q_pos = jnp.pad(query_pos, ((0, 0), (0, 125)))

    lanes = np.arange(D)
    sel = jnp.asarray((lanes[:, None] // DH) == np.arange(16)[None, :],
                      jnp.float32)
    selt = sel.T
    Wg16 = jnp.pad(W_gate, ((0, 0), (0, 16 - H)))
    p16 = jnp.asarray(np.eye(16, D, dtype=np.float32))
    b0 = b_pre_0.reshape(1, LEN)
    b1 = b_pre_1.reshape(1, LEN)

    q = _proj_q(query_x, Wq)
    k, v = _proj_kv(src_x, Wk, Wv)

    ke, ve, qe, pse, pqe = _make_gather(E_tot, NQ, NS0 + NS1, D)(
        e_src, e_dst, k, v, q, src_pos, q_pos)

    msg, ex = _edge_math(qe, ke, ve, pse, pqe,
                         W_pre_0, W_pre_1, b0, b1, Wg16, sel, selt, p16, E)

    scat = _make_scatter(E_tot, NQ, D)
    nparts = scat(e_dst, msg)
    dparts = scat(e_dst, ex)

    return _finalize(query_x, nparts, dparts, selt, Wo)


# speedup vs baseline: 32.0667x; 1.3196x over previous
import functools

import jax
import jax.numpy as jnp
import numpy as np
from jax import lax
from jax.experimental import pallas as pl
from jax.experimental.pallas import tpu as pltpu
from jax.experimental.pallas import tpu_sc as plsc

H = 4
DH = 32
LEN = 32
R0 = 1.0
R1 = 2.0

NC = 2
NS = 16
NW = NC * NS
CH = 80


def _proj_kv_body(x_ref, wk_ref, wv_ref, k_ref, v_ref):
    x = x_ref[...]
    k_ref[...] = jnp.dot(x, wk_ref[...], preferred_element_type=jnp.float32)
    v_ref[...] = jnp.dot(x, wv_ref[...], preferred_element_type=jnp.float32)


def _proj_q_body(x_ref, wq_ref, q_ref):
    q_ref[...] = jnp.dot(x_ref[...], wq_ref[...],
                         preferred_element_type=jnp.float32)


def _proj_kv(src_x, Wk, Wv, blk=3000):
    n = src_x.shape[0]
    d = src_x.shape[1]
    grid = n // blk
    return pl.pallas_call(
        _proj_kv_body,
        grid=(grid,),
        in_specs=[
            pl.BlockSpec((blk, d), lambda i: (i, 0)),
            pl.BlockSpec((d, d), lambda i: (0, 0)),
            pl.BlockSpec((d, d), lambda i: (0, 0)),
        ],
        out_specs=[
            pl.BlockSpec((blk, d), lambda i: (i, 0)),
            pl.BlockSpec((blk, d), lambda i: (i, 0)),
        ],
        out_shape=[
            jax.ShapeDtypeStruct((n, d), jnp.float32),
            jax.ShapeDtypeStruct((n, d), jnp.float32),
        ],
    )(src_x, Wk, Wv)


def _proj_q(query_x, Wq, blk=2000):
    n, d = query_x.shape
    return pl.pallas_call(
        _proj_q_body,
        grid=(n // blk,),
        in_specs=[
            pl.BlockSpec((blk, d), lambda i: (i, 0)),
            pl.BlockSpec((d, d), lambda i: (0, 0)),
        ],
        out_specs=pl.BlockSpec((blk, d), lambda i: (i, 0)),
        out_shape=jax.ShapeDtypeStruct((n, d), jnp.float32),
    )(query_x, Wq)


def _make_gather_compute(E_tot, D, SPAD, QPAD):
    CHG = 128
    n_ci = E_tot // CHG
    nj = (n_ci + NW - 1) // NW
    nt = (nj + 1) // 2
    mesh = plsc.VectorSubcoreMesh(core_axis_name="c", subcore_axis_name="s")
    iot = lambda: lax.iota(jnp.int32, 16)

    @functools.partial(
        pl.kernel,
        mesh=mesh,
        out_type=[
            jax.ShapeDtypeStruct((E_tot // 8, 128), jnp.float32),
            jax.ShapeDtypeStruct((E_tot, D), jnp.float32),
            jax.ShapeDtypeStruct((E_tot, D), jnp.float32),
            jax.ShapeDtypeStruct((E_tot, D), jnp.float32),
        ],
        compiler_params=pltpu.CompilerParams(needs_layout_passes=False),
        scratch_types=(
            [pltpu.VMEM((CHG,), jnp.int32) for _ in range(4)]
            + [pltpu.VMEM((CHG, D), jnp.float32) for _ in range(6)]
            + [pltpu.VMEM((CHG,), jnp.float32) for _ in range(12)]
            + [pltpu.VMEM((16, 128), jnp.float32) for _ in range(2)]
            + [pltpu.SemaphoreType.DMA for _ in range(2)]
        ),
    )
    def gat(e_src_hbm, e_dst_hbm, k_hbm, q_hbm, v_hbm,
            psx_hbm, psy_hbm, psz_hbm, pqx_hbm, pqy_hbm, pqz_hbm,
            pk_out, ke_out, qe_out, ve_out,
            ixs0, ixd0, ixs1, ixd1,
            kb0, qb0, vb0, kb1, qb1, vb1,
            sx0, sy0, sz0, qx0, qy0, qz0,
            sx1, sy1, sz1, qx1, qy1, qz1,
            pk0, pk1,
            sem0, sem1):
        c = lax.axis_index("c")
        s = lax.axis_index("s")
        wid = s * NC + c

        slots = [
            (ixs0, ixd0, kb0, qb0, vb0, sx0, sy0, sz0, qx0, qy0, qz0, pk0, sem0),
            (ixs1, ixd1, kb1, qb1, vb1, sx1, sy1, sz1, qx1, qy1, qz1, pk1, sem1),
        ]

        def issue(j, sl):
            (ixs, ixd, kb, qb, vb, sx, sy, sz, qx, qy, qz, pk, sem) = sl
            ci = j * NW + wid
            base = ci * CHG
            pltpu.sync_copy(e_src_hbm.at[pl.ds(base, CHG)], ixs)
            pltpu.sync_copy(e_dst_hbm.at[pl.ds(base, CHG)], ixd)
            return [
                pltpu.async_copy(k_hbm.at[ixs], kb, sem),
                pltpu.async_copy(q_hbm.at[ixd], qb, sem),
                pltpu.async_copy(v_hbm.at[ixs], vb, sem),
                pltpu.async_copy(psx_hbm.at[ixs], sx, sem),
                pltpu.async_copy(psy_hbm.at[ixs], sy, sem),
                pltpu.async_copy(psz_hbm.at[ixs], sz, sem),
                pltpu.async_copy(pqx_hbm.at[ixd], qx, sem),
                pltpu.async_copy(pqy_hbm.at[ixd], qy, sem),
                pltpu.async_copy(pqz_hbm.at[ixd], qz, sem),
            ]

        def finish(j, sl, cps):
            (ixs, ixd, kb, qb, vb, sx, sy, sz, qx, qy, qz, pk, sem) = sl
            ci = j * NW + wid
            base = ci * CHG
            for cp in cps:
                cp.wait()
            for g in range(CHG // 16):
                rows = iot() + (16 * g)
                cs = pl.ds(16 * g, 16)
                dx = sx[cs] - qx[cs]
                dy = sy[cs] - qy[cs]
                dz = sz[cs] - qz[cs]
                l2 = dx * dx + dy * dy + dz * dz
                plsc.store_scatter(pk, [rows >> 3, ((rows & 7) * 16) + H], l2)
            pltpu.sync_copy(pk, pk_out.at[pl.ds(ci * 16, 16)])
            pltpu.sync_copy(kb, ke_out.at[pl.ds(base, CHG)])
            pltpu.sync_copy(qb, qe_out.at[pl.ds(base, CHG)])
            pltpu.sync_copy(vb, ve_out.at[pl.ds(base, CHG)])

        def body(t, carry):
            j0 = 2 * t
            j1 = 2 * t + 1
            ci0 = j0 * NW + wid
            ci1 = j1 * NW + wid

            @pl.when(ci1 < n_ci)
            def _():
                cps0 = issue(j0, slots[0])
                cps1 = issue(j1, slots[1])
                finish(j0, slots[0], cps0)
                finish(j1, slots[1], cps1)

            @pl.when(jnp.logical_and(ci0 < n_ci, ci1 >= n_ci))
            def _():
                cps0 = issue(j0, slots[0])
                finish(j0, slots[0], cps0)
            return carry

        lax.fori_loop(0, nt, body, 0)

    return gat


def _edge_body(nblk0, rsqrt_dh,
               pk_ref, qe_ref, ke_ref, ve_ref,
               w0_ref, w1_ref, b0_ref, b1_ref, wg_ref,
               sel_ref, selt_ref, p16_ref,
               msg_ref, ex_ref):
    is0 = pl.program_id(0) < nblk0
    r_cut = jnp.where(is0, R0, R1)
    B = ve_ref.shape[0]

    pk = pk_ref[...]
    rep = jnp.reshape(
        jnp.broadcast_to(pk[:, None, :], (B // 8, 8, 128)), (B, 128))
    rowg = lax.broadcasted_iota(jnp.int32, (B, 1), 0) % 8
    lane_r = lax.broadcasted_iota(jnp.int32, (128, 16), 0)
    lane_f = lax.broadcasted_iota(jnp.int32, (128, 16), 1)
    fields = jnp.zeros((B, 16), jnp.float32)
    for g in range(8):
        ug = (lane_r == 16 * g + lane_f).astype(jnp.float32)
        sel_g = jnp.dot(rep, ug, preferred_element_type=jnp.float32)
        fields = fields + jnp.where(rowg == g, sel_g, 0.0)

    fmask = (lax.broadcasted_iota(jnp.int32, (1, 16), 1) == 4).astype(
        jnp.float32)
    fields = fields * fmask
    l2 = fields[:, 4:5]
    length = jnp.sqrt(l2 + 1e-12)
    prod = qe_ref[...] * ke_ref[...]
    qk = jnp.dot(prod, sel_ref[...],
                 preferred_element_type=jnp.float32)

    centers = lax.broadcasted_iota(jnp.int32, (1, LEN), 1).astype(
        jnp.float32) * (r_cut / (LEN - 1))
    sigma = r_cut / LEN
    z = (length - centers) / sigma
    emb = jnp.exp(-0.5 * z * z)

    W_pre = jnp.where(is0, w0_ref[...], w1_ref[...])
    b_pre = jnp.where(is0, b0_ref[...], b1_ref[...])
    pre = jnp.dot(emb, W_pre, preferred_element_type=jnp.float32) + b_pre
    es = pre * jax.nn.sigmoid(pre)
    gate = jnp.dot(es, wg_ref[...], preferred_element_type=jnp.float32)

    env = 0.5 * (jnp.cos(np.pi * jnp.clip(length / r_cut, 0.0, 1.0)) + 1.0)
    ex = jnp.exp(qk * rsqrt_dh + gate) * (env + 1e-6)

    ex_ref[...] = jnp.dot(ex, p16_ref[...],
                          preferred_element_type=jnp.float32)
    msg_ref[...] = jnp.dot(ex, selt_ref[...],
                           preferred_element_type=jnp.float32) * ve_ref[...]


def _edge_math(pk, qe, ke, ve, W0, W1, b0, b1, Wg16, sel, selt, p16,
               E_per_scale, blk=6400):
    E_tot = ve.shape[0]
    D = ve.shape[1]
    nblk0 = E_per_scale // blk
    grid = E_tot // blk
    body = functools.partial(_edge_body, nblk0, float(1.0 / np.sqrt(DH)))
    row = lambda i: (i, 0)
    full = lambda i: (0, 0)
    return pl.pallas_call(
        body,
        grid=(grid,),
        in_specs=[
            pl.BlockSpec((blk // 8, 128), row),
            pl.BlockSpec((blk, D), row),
            pl.BlockSpec((blk, D), row),
            pl.BlockSpec((blk, D), row),
            pl.BlockSpec((LEN, LEN), full),
            pl.BlockSpec((LEN, LEN), full),
            pl.BlockSpec((1, LEN), full),
            pl.BlockSpec((1, LEN), full),
            pl.BlockSpec((LEN, 16), full),
            pl.BlockSpec((D, 16), full),
            pl.BlockSpec((16, D), full),
            pl.BlockSpec((16, D), full),
        ],
        out_specs=[
            pl.BlockSpec((blk, D), row),
            pl.BlockSpec((blk, D), row),
        ],
        out_shape=[
            jax.ShapeDtypeStruct((E_tot, D), jnp.float32),
            jax.ShapeDtypeStruct((E_tot, D), jnp.float32),
        ],
    )(pk, qe, ke, ve, W0, W1, b0, b1, Wg16, sel, selt, p16)


def _make_scatter(E_tot, NQ, D):
    n_ci = E_tot // CH
    nj = n_ci // NS
    ZR = 200
    n_zc = NQ // ZR
    mesh = plsc.VectorSubcoreMesh(core_axis_name="c", subcore_axis_name="s")

    @functools.partial(
        pl.kernel,
        mesh=mesh,
        out_type=[
            jax.ShapeDtypeStruct((NQ, D), jnp.float32),
            jax.ShapeDtypeStruct((NQ, D), jnp.float32),
        ],
        scratch_types=[
            pltpu.VMEM((CH,), jnp.int32),
            pltpu.VMEM((CH,), jnp.int32),
            pltpu.VMEM((CH, D), jnp.float32),
            pltpu.VMEM((CH, D), jnp.float32),
            pltpu.VMEM((ZR, D), jnp.float32),
            pltpu.VMEM_SHARED((NQ, D), jnp.float32),
            pltpu.SemaphoreType.DMA,
            pltpu.SemaphoreType.DMA,
        ],
    )
    def scatter(e_dst_hbm, msg_hbm, ex_hbm, num_out, den_out,
                idx_v0, idx_v1, vbuf0, vbuf1, zrow, sh, sem0, sem1):
        c = lax.axis_index("c")
        s = lax.axis_index("s")

        zero16 = jnp.zeros((16,), jnp.float32)

        def zb(i, carry):
            r = i // (D // 16)
            col = (i % (D // 16)) * 16
            zrow[r, pl.ds(col, 16)] = zero16
            return carry
        lax.fori_loop(0, ZR * (D // 16), zb, 0)

        def zfill(t, carry):
            cid = s + NS * t

            @pl.when(cid < n_zc)
            def _():
                pltpu.sync_copy(zrow, sh.at[pl.ds(cid * ZR, ZR)])
            return carry
        lax.fori_loop(0, (n_zc + NS - 1) // NS, zfill, 0)

        plsc.subcore_barrier()

        def make_body(val_hbm):
            def issue(j, idx_v, vbuf, sem):
                base = (j * NS + s) * CH
                return [
                    pltpu.async_copy(e_dst_hbm.at[pl.ds(base, CH)], idx_v,
                                     sem),
                    pltpu.async_copy(val_hbm.at[pl.ds(base, CH)], vbuf, sem),
                ]

            def body(t, carry):
                cps0 = issue(2 * t, idx_v0, vbuf0, sem0)
                cps1 = issue(2 * t + 1, idx_v1, vbuf1, sem1)
                for cp in cps0:
                    cp.wait()
                pltpu.sync_copy(vbuf0, sh.at[idx_v0], add=True)
                for cp in cps1:
                    cp.wait()
                pltpu.sync_copy(vbuf1, sh.at[idx_v1], add=True)
                return carry
            return body

        @pl.when(c == 0)
        def _():
            lax.fori_loop(0, nj // 2, make_body(msg_hbm), 0)

        @pl.when(c == 1)
        def _():
            lax.fori_loop(0, nj // 2, make_body(ex_hbm), 0)

        plsc.subcore_barrier()

        def wb(t, carry):
            cid = s + NS * t

            @pl.when(cid < n_zc)
            def _():
                r0 = cid * ZR

                @pl.when(c == 0)
                def _():
                    pltpu.sync_copy(sh.at[pl.ds(r0, ZR)],
                                    num_out.at[pl.ds(r0, ZR)])

                @pl.when(c == 1)
                def _():
                    pltpu.sync_copy(sh.at[pl.ds(r0, ZR)],
                                    den_out.at[pl.ds(r0, ZR)])
            return carry
        lax.fori_loop(0, (n_zc + NS - 1) // NS, wb, 0)

    return scatter


def _final_body(qx_ref, n_ref, d_ref, s16_ref, wo_ref, out_ref):
    den = jnp.dot(d_ref[...][:, :16], s16_ref[...],
                  preferred_element_type=jnp.float32) + 1e-9
    agg = n_ref[...] / den
    out_ref[...] = jnp.dot(agg, wo_ref[...],
                           preferred_element_type=jnp.float32) + qx_ref[...]


def _finalize(query_x, num, den, S16, Wo, blk=2000):
    NQ, D = query_x.shape
    nb = NQ // blk
    return pl.pallas_call(
        _final_body,
        grid=(nb,),
        in_specs=[
            pl.BlockSpec((blk, D), lambda i: (i, 0)),
            pl.BlockSpec((blk, D), lambda i: (i, 0)),
            pl.BlockSpec((blk, D), lambda i: (i, 0)),
            pl.BlockSpec((16, D), lambda i: (0, 0)),
            pl.BlockSpec((D, D), lambda i: (0, 0)),
        ],
        out_specs=pl.BlockSpec((blk, D), lambda i: (i, 0)),
        out_shape=jax.ShapeDtypeStruct((NQ, D), jnp.float32),
    )(query_x, num, den, S16, Wo)


def kernel(query_x, query_pos, src_x_0, src_pos_0, src_x_1, src_pos_1,
           edge_src_0, edge_dst_0, edge_src_1, edge_dst_1,
           W_pre_0, b_pre_0, W_pre_1, b_pre_1, Wq, Wk, Wv, Wo, W_gate):
    NQ, D = query_x.shape
    NS0 = src_x_0.shape[0]
    NS1 = src_x_1.shape[0]
    E = edge_src_0.shape[0]
    E_tot = 2 * E

    src_x = jnp.concatenate([src_x_0, src_x_1], axis=0)
    e_src = jnp.concatenate([edge_src_0, edge_src_1 + NS0], axis=0)
    e_dst = jnp.concatenate([edge_dst_0, edge_dst_1], axis=0)
    NSRC = NS0 + NS1
    SPAD = ((NSRC + NS * 8 - 1) // (NS * 8)) * (NS * 8)
    QPAD = ((NQ + NS * 8 - 1) // (NS * 8)) * (NS * 8)
    src_pos = jnp.concatenate([src_pos_0, src_pos_1], axis=0)
    psx = jnp.pad(src_pos[:, 0], (0, SPAD - NSRC))
    psy = jnp.pad(src_pos[:, 1], (0, SPAD - NSRC))
    psz = jnp.pad(src_pos[:, 2], (0, SPAD - NSRC))
    pqx = jnp.pad(query_pos[:, 0], (0, QPAD - NQ))
    pqy = jnp.pad(query_pos[:, 1], (0, QPAD - NQ))
    pqz = jnp.pad(query_pos[:, 2], (0, QPAD - NQ))

    lanes = np.arange(D)
    sel = jnp.asarray((lanes[:, None] // DH) == np.arange(16)[None, :],
                      jnp.float32)
    selt = sel.T
    Wg16 = jnp.pad(W_gate, ((0, 0), (0, 16 - H)))
    p16 = jnp.asarray(np.eye(16, D, dtype=np.float32))
    b0 = b_pre_0.reshape(1, LEN)
    b1 = b_pre_1.reshape(1, LEN)

    q = _proj_q(query_x, Wq)
    k, v = _proj_kv(src_x, Wk, Wv)

    pk, ke, qe, ve = _make_gather_compute(E_tot, D, SPAD, QPAD)(
        e_src, e_dst, k, q, v, psx, psy, psz, pqx, pqy, pqz)

    msg, ex = _edge_math(pk, qe, ke, ve, W_pre_0, W_pre_1, b0, b1, Wg16,
                         sel, selt, p16, E)

    num, den = _make_scatter(E_tot, NQ, D)(e_dst, msg, ex)

    return _finalize(query_x, num, den, selt, Wo)
